# baseline (speedup 1.0000x reference)
import functools

import jax
import jax.numpy as jnp
from jax import lax
from jax.experimental import pallas as pl
from jax.experimental.pallas import tpu as pltpu
from jax.experimental.pallas import tpu_sc as plsc

B = 16384
C = 10
V = 1_000_000
NC = 2
NS = 16
NW = NC * NS
RPW = B // NW
CHUNK = 128
NCH = RPW // CHUNK

BETA = 0.7
LAMBDA_ = 3.0
CLIP_LO = 0.0001
CLIP_HI = 1.0 - 0.0001

_sc_mesh = plsc.VectorSubcoreMesh(core_axis_name="c", subcore_axis_name="s")


@functools.partial(
    pl.kernel,
    mesh=_sc_mesh,
    compiler_params=pltpu.CompilerParams(use_tc_tiling_on_sc=False),
    out_type=jax.ShapeDtypeStruct((C, NW * NCH, CHUNK), jnp.float32),
    scratch_types=[
        pltpu.VMEM((NCH, CHUNK), jnp.int32),
        pltpu.VMEM((C * NCH, CHUNK), jnp.int32),
        pltpu.VMEM((C, NCH, CHUNK), jnp.float32),
        pltpu.SemaphoreType.DMA,
    ],
)
def _sc_gather(idx_hbm, tflat_hbm, out_hbm, idx_v, addr_v, vals_v, sem):
    wid = lax.axis_index("s") * NC + lax.axis_index("c")
    pltpu.sync_copy(idx_hbm.at[pl.ds(wid * NCH, NCH)], idx_v)
    for c in range(C):
        for ch in range(NCH):
            for g in range(CHUNK // 16):
                iv = idx_v[ch, pl.ds(g * 16, 16)]
                addr_v[c * NCH + ch, pl.ds(g * 16, 16)] = iv + (c * V)
    copies = [
        pltpu.async_copy(
            tflat_hbm.at[addr_v.at[c * NCH + ch]],
            vals_v.at[c, ch],
            sem,
        )
        for c in range(C)
        for ch in range(NCH)
    ]
    for cp in copies:
        cp.wait()
    for c in range(C):
        pltpu.sync_copy(vals_v.at[c], out_hbm.at[c, pl.ds(wid * NCH, NCH)])


def _tc_loss_body(out_t_ref, lab_ref, trow_t_ref, loss_ref):
    x = out_t_ref[...]
    m = jnp.max(x, axis=0, keepdims=True)
    e = jnp.exp(x - m)
    s = jnp.sum(e, axis=0, keepdims=True)
    y = jnp.clip(e / s, CLIP_LO, CLIP_HI)
    ysum = jnp.sum(y, axis=0, keepdims=True)
    nr = BETA * trow_t_ref[...] + (1.0 - BETA) * (y / ysum)
    dot = jnp.sum(nr * y, axis=0)
    lab = lab_ref[0, :]
    cls = lax.broadcasted_iota(jnp.int32, x.shape, 0)
    xl = jnp.sum(jnp.where(cls == lab[None, :], x, 0.0), axis=0)
    ce = (m[0] + jnp.log(s[0])) - xl
    elr = jnp.log(1.0 - dot)
    loss_ref[0, 0] = jnp.mean(ce) + LAMBDA_ * jnp.mean(elr)


_tc_loss = pl.pallas_call(
    _tc_loss_body,
    out_shape=jax.ShapeDtypeStruct((1, 1), jnp.float32),
    out_specs=pl.BlockSpec(memory_space=pltpu.SMEM),
)


def kernel(index, output, label, target):
    tflat = jnp.reshape(target.T, (V * C,))
    trow_t = _sc_gather(index.reshape(NW * NCH, CHUNK), tflat)
    loss = _tc_loss(output.T, label[None, :], trow_t.reshape(C, B))
    return loss[0, 0]


# speedup vs baseline: 1.0013x; 1.0013x over previous
import functools

import jax
import jax.numpy as jnp
from jax import lax
from jax.experimental import pallas as pl
from jax.experimental.pallas import tpu as pltpu
from jax.experimental.pallas import tpu_sc as plsc

B = 16384
C = 10
V = 1_000_000
NC = 2
NS = 16
NW = NC * NS
RPW = B // NW
CHUNK = 128
NCH = RPW // CHUNK

BETA = 0.7
LAMBDA_ = 3.0
CLIP_LO = 0.0001
CLIP_HI = 1.0 - 0.0001

_sc_mesh = plsc.VectorSubcoreMesh(core_axis_name="c", subcore_axis_name="s")


@functools.partial(
    pl.kernel,
    mesh=_sc_mesh,
    compiler_params=pltpu.CompilerParams(use_tc_tiling_on_sc=False),
    out_type=jax.ShapeDtypeStruct((C, NW * NCH, CHUNK), jnp.float32),
    scratch_types=[
        pltpu.VMEM((NCH, CHUNK), jnp.int32),
        pltpu.VMEM((C, NCH, CHUNK), jnp.float32),
        pltpu.SemaphoreType.DMA,
    ],
)
def _sc_gather(idx_hbm, tgt_t_hbm, out_hbm, idx_v, vals_v, sem):
    wid = lax.axis_index("s") * NC + lax.axis_index("c")
    pltpu.sync_copy(idx_hbm.at[pl.ds(wid * NCH, NCH)], idx_v)
    copies = [
        pltpu.async_copy(
            tgt_t_hbm.at[c].at[idx_v.at[ch]],
            vals_v.at[c, ch],
            sem,
        )
        for c in range(C)
        for ch in range(NCH)
    ]
    for cp in copies:
        cp.wait()
    for c in range(C):
        pltpu.sync_copy(vals_v.at[c], out_hbm.at[c, pl.ds(wid * NCH, NCH)])


def _tc_loss_body(out_t_ref, lab_ref, trow_t_ref, loss_ref):
    x = out_t_ref[...]
    m = jnp.max(x, axis=0, keepdims=True)
    e = jnp.exp(x - m)
    s = jnp.sum(e, axis=0, keepdims=True)
    y = jnp.clip(e / s, CLIP_LO, CLIP_HI)
    ysum = jnp.sum(y, axis=0, keepdims=True)
    nr = BETA * trow_t_ref[...] + (1.0 - BETA) * (y / ysum)
    dot = jnp.sum(nr * y, axis=0)
    lab = lab_ref[0, :]
    cls = lax.broadcasted_iota(jnp.int32, x.shape, 0)
    xl = jnp.sum(jnp.where(cls == lab[None, :], x, 0.0), axis=0)
    ce = (m[0] + jnp.log(s[0])) - xl
    elr = jnp.log(1.0 - dot)
    loss_ref[0, 0] = jnp.mean(ce) + LAMBDA_ * jnp.mean(elr)


_tc_loss = pl.pallas_call(
    _tc_loss_body,
    out_shape=jax.ShapeDtypeStruct((1, 1), jnp.float32),
    out_specs=pl.BlockSpec(memory_space=pltpu.SMEM),
)


def kernel(index, output, label, target):
    trow_t = _sc_gather(index.reshape(NW * NCH, CHUNK), target.T)
    loss = _tc_loss(output.T, label[None, :], trow_t.reshape(C, B))
    return loss[0, 0]


# speedup vs baseline: 11.0428x; 11.0284x over previous
import functools

import jax
import jax.numpy as jnp
from jax import lax
from jax.experimental import pallas as pl
from jax.experimental.pallas import tpu as pltpu
from jax.experimental.pallas import tpu_sc as plsc

B = 16384
C = 10
V = 1_000_000
NC = 2
NS = 16
NW = NC * NS
RPW = B // NW
CHUNK = 128
NCH = RPW // CHUNK

NT = 7813
NFULL = 7808
TPW = NFULL // NW
CH_T = 61
NCHK = TPW // CH_T

BETA = 0.7
LAMBDA_ = 3.0
CLIP_LO = 0.0001
CLIP_HI = 1.0 - 0.0001

_sc_mesh = plsc.VectorSubcoreMesh(core_axis_name="c", subcore_axis_name="s")


@functools.partial(
    pl.kernel,
    mesh=_sc_mesh,
    compiler_params=pltpu.CompilerParams(use_tc_tiling_on_sc=True),
    out_type=jax.ShapeDtypeStruct((2 * NT, 8, 128), jnp.float32),
    scratch_types=[
        pltpu.VMEM((CH_T, 10, 128), jnp.float32),
        pltpu.SemaphoreType.DMA,
        pltpu.SemaphoreType.DMA,
    ],
)
def _sc_depad(tgt_t_hbm, out_hbm, buf, in_sem, out_sem):
    wid = lax.axis_index("s") * NC + lax.axis_index("c")
    t_base = wid * TPW
    for k in range(NCHK):
        t0 = t_base + k * CH_T
        ins = [
            pltpu.async_copy(
                tgt_t_hbm.at[:, pl.ds((t0 + j) * 128, 128)], buf.at[j], in_sem)
            for j in range(CH_T)
        ]
        for cp in ins:
            cp.wait()
        o0 = pltpu.async_copy(
            buf.at[:, pl.ds(0, 8)], out_hbm.at[pl.ds(t0, CH_T)], out_sem)
        o1 = pltpu.async_copy(
            buf.at[:, pl.ds(8, 2)],
            out_hbm.at[pl.ds(NT + t0, CH_T), pl.ds(0, 2)], out_sem)
        o0.wait()
        o1.wait()
    @pl.when(wid < 4)
    def _():
        t = NFULL + wid
        pltpu.async_copy(
            tgt_t_hbm.at[:, pl.ds(t * 128, 128)], buf.at[0], in_sem).wait()
        pltpu.async_copy(
            buf.at[0, pl.ds(0, 8)], out_hbm.at[t], out_sem).wait()
        pltpu.async_copy(
            buf.at[0, pl.ds(8, 2)], out_hbm.at[NT + t, pl.ds(0, 2)],
            out_sem).wait()


@functools.partial(
    pl.kernel,
    mesh=_sc_mesh,
    compiler_params=pltpu.CompilerParams(use_tc_tiling_on_sc=False),
    out_type=jax.ShapeDtypeStruct((C, NW * NCH, CHUNK), jnp.float32),
    scratch_types=[
        pltpu.VMEM((NCH, CHUNK), jnp.int32),
        pltpu.VMEM((C * NCH, CHUNK), jnp.int32),
        pltpu.VMEM((C, NCH, CHUNK), jnp.float32),
        pltpu.SemaphoreType.DMA,
    ],
)
def _sc_gather(idx_hbm, tflat_hbm, out_hbm, idx_v, addr_v, vals_v, sem):
    wid = lax.axis_index("s") * NC + lax.axis_index("c")
    pltpu.sync_copy(idx_hbm.at[pl.ds(wid * NCH, NCH)], idx_v)
    for ch in range(NCH):
        for g in range(CHUNK // 16):
            iv = idx_v[ch, pl.ds(g * 16, 16)]
            w0 = ((iv >> 7) << 10) | (iv & 127)
            for c in range(C):
                kc = (c // 8) * (NT * 1024) + (c % 8) * 128
                addr_v[c * NCH + ch, pl.ds(g * 16, 16)] = w0 + kc
    copies = [
        pltpu.async_copy(
            tflat_hbm.at[addr_v.at[c * NCH + ch]], vals_v.at[c, ch], sem)
        for c in range(C)
        for ch in range(NCH)
    ]
    for cp in copies:
        cp.wait()
    for c in range(C):
        pltpu.sync_copy(vals_v.at[c], out_hbm.at[c, pl.ds(wid * NCH, NCH)])


TAIL0 = NFULL * 128 + 4 * 128


def _tc_loss_body(out_t_ref, lab_ref, idx_ref, trow_t_ref, tgt_any_ref,
                  loss_ref, tail_v, tail_sem):
    pltpu.make_async_copy(
        tgt_any_ref.at[:, pl.ds(TAIL0, 64)], tail_v, tail_sem).start()
    x = out_t_ref[...]
    m = jnp.max(x, axis=0, keepdims=True)
    e = jnp.exp(x - m)
    s = jnp.sum(e, axis=0, keepdims=True)
    y = jnp.clip(e / s, CLIP_LO, CLIP_HI)
    ysum = jnp.sum(y, axis=0, keepdims=True)
    pltpu.make_async_copy(
        tgt_any_ref.at[:, pl.ds(TAIL0, 64)], tail_v, tail_sem).wait()
    d = idx_ref[0, :] - TAIL0
    oh = (lax.broadcasted_iota(jnp.int32, (64, B), 0) == d[None, :]).astype(
        jnp.float32)
    t_sel = jax.lax.dot_general(
        tail_v[...], oh, (((1,), (0,)), ((), ())),
        preferred_element_type=jnp.float32)
    tr = jnp.where((d >= 0)[None, :], t_sel, trow_t_ref[...])
    nr = BETA * tr + (1.0 - BETA) * (y / ysum)
    dot = jnp.sum(nr * y, axis=0)
    lab = lab_ref[0, :]
    cls = lax.broadcasted_iota(jnp.int32, x.shape, 0)
    xl = jnp.sum(jnp.where(cls == lab[None, :], x, 0.0), axis=0)
    ce = (m[0] + jnp.log(s[0])) - xl
    elr = jnp.log(1.0 - dot)
    loss_ref[0, 0] = jnp.mean(ce) + LAMBDA_ * jnp.mean(elr)


_tc_loss = pl.pallas_call(
    _tc_loss_body,
    out_shape=jax.ShapeDtypeStruct((1, 1), jnp.float32),
    in_specs=[
        pl.BlockSpec(memory_space=pltpu.VMEM),
        pl.BlockSpec(memory_space=pltpu.VMEM),
        pl.BlockSpec(memory_space=pltpu.VMEM),
        pl.BlockSpec(memory_space=pltpu.VMEM),
        pl.BlockSpec(memory_space=pl.ANY),
    ],
    out_specs=pl.BlockSpec(memory_space=pltpu.SMEM),
    scratch_shapes=[
        pltpu.VMEM((C, 64), jnp.float32),
        pltpu.SemaphoreType.DMA,
    ],
)


def kernel(index, output, label, target):
    t_tiles = _sc_depad(target.T)
    tflat = jnp.reshape(t_tiles, (2 * NT * 8 * 128,))
    trow_t = _sc_gather(index.reshape(NW * NCH, CHUNK), tflat)
    loss = _tc_loss(output.T, label[None, :], index[None, :],
                    trow_t.reshape(C, B), target.T)
    return loss[0, 0]


# speedup vs baseline: 11.0928x; 1.0045x over previous
import functools

import jax
import jax.numpy as jnp
from jax import lax
from jax.experimental import pallas as pl
from jax.experimental.pallas import tpu as pltpu
from jax.experimental.pallas import tpu_sc as plsc

B = 16384
C = 10
V = 1_000_000
NC = 2
NS = 16
NW = NC * NS
RPW = B // NW
CHUNK = 128
NCH = RPW // CHUNK

NT = 7813
NFULL = 7808
TPW = NFULL // NW
CH_T = 30
NCHK = 8
REM_T = TPW - NCHK * CH_T

BETA = 0.7
LAMBDA_ = 3.0
CLIP_LO = 0.0001
CLIP_HI = 1.0 - 0.0001

_sc_mesh = plsc.VectorSubcoreMesh(core_axis_name="c", subcore_axis_name="s")


@functools.partial(
    pl.kernel,
    mesh=_sc_mesh,
    compiler_params=pltpu.CompilerParams(use_tc_tiling_on_sc=True),
    out_type=jax.ShapeDtypeStruct((2 * NT, 8, 128), jnp.float32),
    scratch_types=[
        pltpu.VMEM((2, CH_T, 10, 128), jnp.float32),
        pltpu.SemaphoreType.DMA,
        pltpu.SemaphoreType.DMA,
        pltpu.SemaphoreType.DMA,
        pltpu.SemaphoreType.DMA,
    ],
)
def _sc_depad(tgt_t_hbm, out_hbm, buf, in_sem0, in_sem1, out_sem0, out_sem1):
    wid = lax.axis_index("s") * NC + lax.axis_index("c")
    t_base = wid * TPW
    in_sems = (in_sem0, in_sem1)
    out_sems = (out_sem0, out_sem1)
    ins_h = [None, None]
    outs_h = [None, None]
    chunk_t0 = [None, None]

    def fire_outs(p):
        t0 = chunk_t0[p]
        for cp in ins_h[p]:
            cp.wait()
        outs_h[p] = [
            pltpu.async_copy(
                buf.at[p, :, pl.ds(0, 8)], out_hbm.at[pl.ds(t0, CH_T)],
                out_sems[p]),
            pltpu.async_copy(
                buf.at[p, :, pl.ds(8, 2)],
                out_hbm.at[pl.ds(NT + t0, CH_T), pl.ds(0, 2)], out_sems[p]),
        ]

    for k in range(NCHK):
        p = k & 1
        if outs_h[p] is not None:
            for cp in outs_h[p]:
                cp.wait()
        t0 = t_base + k * CH_T
        chunk_t0[p] = t0
        ins_h[p] = [
            pltpu.async_copy(
                tgt_t_hbm.at[:, pl.ds((t0 + j) * 128, 128)],
                buf.at[p, j], in_sems[p])
            for j in range(CH_T)
        ]
        q = 1 - p
        if k >= 1:
            fire_outs(q)
    fire_outs((NCHK - 1) & 1)
    for p in range(2):
        if outs_h[p] is not None:
            for cp in outs_h[p]:
                cp.wait()
    r0 = t_base + NCHK * CH_T
    rins = [
        pltpu.async_copy(
            tgt_t_hbm.at[:, pl.ds((r0 + j) * 128, 128)], buf.at[0, j],
            in_sem0)
        for j in range(REM_T)
    ]
    for cp in rins:
        cp.wait()
    pltpu.async_copy(
        buf.at[0, pl.ds(0, REM_T), pl.ds(0, 8)],
        out_hbm.at[pl.ds(r0, REM_T)], out_sem0).wait()
    pltpu.async_copy(
        buf.at[0, pl.ds(0, REM_T), pl.ds(8, 2)],
        out_hbm.at[pl.ds(NT + r0, REM_T), pl.ds(0, 2)], out_sem0).wait()
    @pl.when(wid < 4)
    def _():
        t = NFULL + wid
        pltpu.async_copy(
            tgt_t_hbm.at[:, pl.ds(t * 128, 128)], buf.at[0, 0], in_sem0).wait()
        pltpu.async_copy(
            buf.at[0, 0, pl.ds(0, 8)], out_hbm.at[t], out_sem0).wait()
        pltpu.async_copy(
            buf.at[0, 0, pl.ds(8, 2)], out_hbm.at[NT + t, pl.ds(0, 2)],
            out_sem0).wait()


@functools.partial(
    pl.kernel,
    mesh=_sc_mesh,
    compiler_params=pltpu.CompilerParams(use_tc_tiling_on_sc=False),
    out_type=jax.ShapeDtypeStruct((C, NW * NCH, CHUNK), jnp.float32),
    scratch_types=[
        pltpu.VMEM((NCH, CHUNK), jnp.int32),
        pltpu.VMEM((C * NCH, CHUNK), jnp.int32),
        pltpu.VMEM((C, NCH, CHUNK), jnp.float32),
        pltpu.SemaphoreType.DMA,
    ],
)
def _sc_gather(idx_hbm, tflat_hbm, out_hbm, idx_v, addr_v, vals_v, sem):
    wid = lax.axis_index("s") * NC + lax.axis_index("c")
    pltpu.sync_copy(idx_hbm.at[pl.ds(wid * NCH, NCH)], idx_v)
    for ch in range(NCH):
        for g in range(CHUNK // 16):
            iv = idx_v[ch, pl.ds(g * 16, 16)]
            w0 = ((iv >> 7) << 10) | (iv & 127)
            for c in range(C):
                kc = (c // 8) * (NT * 1024) + (c % 8) * 128
                addr_v[c * NCH + ch, pl.ds(g * 16, 16)] = w0 + kc
    copies = [
        pltpu.async_copy(
            tflat_hbm.at[addr_v.at[c * NCH + ch]], vals_v.at[c, ch], sem)
        for c in range(C)
        for ch in range(NCH)
    ]
    for cp in copies:
        cp.wait()
    for c in range(C):
        pltpu.sync_copy(vals_v.at[c], out_hbm.at[c, pl.ds(wid * NCH, NCH)])


TAIL0 = NFULL * 128 + 4 * 128


def _tc_loss_body(out_t_ref, lab_ref, idx_ref, trow_t_ref, tgt_any_ref,
                  loss_ref, tail_v, tail_sem):
    pltpu.make_async_copy(
        tgt_any_ref.at[:, pl.ds(TAIL0, 64)], tail_v, tail_sem).start()
    x = out_t_ref[...]
    m = jnp.max(x, axis=0, keepdims=True)
    e = jnp.exp(x - m)
    s = jnp.sum(e, axis=0, keepdims=True)
    y = jnp.clip(e / s, CLIP_LO, CLIP_HI)
    ysum = jnp.sum(y, axis=0, keepdims=True)
    pltpu.make_async_copy(
        tgt_any_ref.at[:, pl.ds(TAIL0, 64)], tail_v, tail_sem).wait()
    d = idx_ref[0, :] - TAIL0
    oh = (lax.broadcasted_iota(jnp.int32, (64, B), 0) == d[None, :]).astype(
        jnp.float32)
    t_sel = jax.lax.dot_general(
        tail_v[...], oh, (((1,), (0,)), ((), ())),
        preferred_element_type=jnp.float32)
    tr = jnp.where((d >= 0)[None, :], t_sel, trow_t_ref[...])
    nr = BETA * tr + (1.0 - BETA) * (y / ysum)
    dot = jnp.sum(nr * y, axis=0)
    lab = lab_ref[0, :]
    cls = lax.broadcasted_iota(jnp.int32, x.shape, 0)
    xl = jnp.sum(jnp.where(cls == lab[None, :], x, 0.0), axis=0)
    ce = (m[0] + jnp.log(s[0])) - xl
    elr = jnp.log(1.0 - dot)
    loss_ref[0, 0] = jnp.mean(ce) + LAMBDA_ * jnp.mean(elr)


_tc_loss = pl.pallas_call(
    _tc_loss_body,
    out_shape=jax.ShapeDtypeStruct((1, 1), jnp.float32),
    in_specs=[
        pl.BlockSpec(memory_space=pltpu.VMEM),
        pl.BlockSpec(memory_space=pltpu.VMEM),
        pl.BlockSpec(memory_space=pltpu.VMEM),
        pl.BlockSpec(memory_space=pltpu.VMEM),
        pl.BlockSpec(memory_space=pl.ANY),
    ],
    out_specs=pl.BlockSpec(memory_space=pltpu.SMEM),
    scratch_shapes=[
        pltpu.VMEM((C, 64), jnp.float32),
        pltpu.SemaphoreType.DMA,
    ],
)


def kernel(index, output, label, target):
    t_tiles = _sc_depad(target.T)
    tflat = jnp.reshape(t_tiles, (2 * NT * 8 * 128,))
    trow_t = _sc_gather(index.reshape(NW * NCH, CHUNK), tflat)
    loss = _tc_loss(output.T, label[None, :], index[None, :],
                    trow_t.reshape(C, B), target.T)
    return loss[0, 0]
